# baseline (speedup 1.0000x reference)
import functools

import jax
import jax.numpy as jnp
from jax import lax
from jax.experimental import pallas as pl
from jax.experimental.pallas import tpu as pltpu
from jax.experimental.pallas import tpu_sc as plsc

N_NODES = 10000
F = 128
CH = 64
NBUF = 3
NC = 2
NS = 16
NW = NC * NS
STRIPE = 624
TAIL = N_NODES - NS * STRIPE


def _mm_body(x_ref, w_ref, o_ref):
    o_ref[...] = jnp.dot(x_ref[...], w_ref[...],
                         preferred_element_type=jnp.float32)


def _matmul(x, w):
    m = x.shape[0]
    bm = 1000
    grid = (m // bm,)
    return pl.pallas_call(
        _mm_body,
        grid=grid,
        in_specs=[
            pl.BlockSpec((bm, F), lambda i: (i, 0)),
            pl.BlockSpec((F, F), lambda i: (0, 0)),
        ],
        out_specs=pl.BlockSpec((bm, F), lambda i: (i, 0)),
        out_shape=jax.ShapeDtypeStruct((m, F), jnp.float32),
    )(x, w)


def _combine_body(p_ref, b_ref, o_ref):
    o_ref[...] = p_ref[0] + p_ref[1] + b_ref[...]


def _combine(parts, bias2d):
    m = parts.shape[1]
    bm = 1000
    grid = (m // bm,)
    return pl.pallas_call(
        _combine_body,
        grid=grid,
        in_specs=[
            pl.BlockSpec((NC, bm, F), lambda i: (0, i, 0)),
            pl.BlockSpec((1, F), lambda i: (0, 0)),
        ],
        out_specs=pl.BlockSpec((bm, F), lambda i: (i, 0)),
        out_shape=jax.ShapeDtypeStruct((m, F), jnp.float32),
    )(parts, bias2d)


def _sc_spmm(support, cols3, rpair3, vals3, n_chunks):
    mesh = plsc.VectorSubcoreMesh(core_axis_name="c", subcore_axis_name="s")

    @functools.partial(
        pl.kernel,
        mesh=mesh,
        out_type=jax.ShapeDtypeStruct((NC, N_NODES, F), jnp.float32),
        scratch_types=[
            pltpu.VMEM((n_chunks // 2, 2 * CH), jnp.int32),
            pltpu.VMEM((n_chunks // 4, 2 * CH), jnp.int32),
            pltpu.VMEM((n_chunks // 4, 2 * CH), jnp.int32),
            pltpu.VMEM((NBUF, CH), jnp.int32),
            pltpu.VMEM((NBUF, CH, F), jnp.float32),
            pltpu.VMEM_SHARED((N_NODES, F), jnp.float32),
            pltpu.SemaphoreType.DMA,
            pltpu.SemaphoreType.DMA,
            pltpu.SemaphoreType.DMA,
        ],
    )
    def k(support_hbm, cols_hbm, rows_hbm, vals_hbm, out_hbm,
          cols_v, rows_v, vals_v, rbuf, gbuf, acc, sg0, sg1, sg2):
        c = lax.axis_index("c")
        s = lax.axis_index("s")
        wid = c * NS + s
        gsems = (sg0, sg1, sg2)

        pltpu.sync_copy(cols_hbm.at[wid], cols_v)
        pltpu.sync_copy(rows_hbm.at[wid], rows_v)
        pltpu.sync_copy(vals_hbm.at[wid], vals_v)

        zeros16 = jnp.zeros((16,), jnp.float32)

        def zrow(e, carry):
            for j in range(F // 16):
                gbuf[0, e, pl.ds(j * 16, 16)] = zeros16
            return carry

        lax.fori_loop(0, CH, zrow, 0)
        base = s * STRIPE
        for t in range(STRIPE // CH):
            pltpu.sync_copy(gbuf.at[0], acc.at[pl.ds(base + t * CH, CH)])
        rem = STRIPE % CH
        if rem:
            pltpu.sync_copy(gbuf.at[0, pl.ds(0, rem)],
                            acc.at[pl.ds(base + STRIPE - rem, rem)])

        @pl.when(s == 0)
        def _():
            pltpu.sync_copy(gbuf.at[0, pl.ds(0, TAIL)],
                            acc.at[pl.ds(NS * STRIPE, TAIL)])

        plsc.subcore_barrier()

        def g_desc(t, b):
            return pltpu.make_async_copy(
                support_hbm.at[cols_v.at[t // 2, pl.ds((t % 2) * CH, CH)]],
                gbuf.at[b], gsems[b])

        def unpack_rows(t, dst):
            co = (t % 4) * (CH // 2)
            for jg in range(2):
                p = rows_v[t // 4, pl.ds(co + jg * 16, 16)]
                rbuf[dst, pl.ds(jg * 32, 16)] = p & 0xFFFF
                rbuf[dst, pl.ds(jg * 32 + 16, 16)] = (
                    lax.shift_right_logical(p, 16))

        g_desc(0, 0).start()
        g_desc(1, 1).start()

        def triple(kk, carry):
            for b in range(NBUF):
                k_ = kk * NBUF + b
                g_desc(k_, b).wait()

                def scale(g, cc):
                    vi = vals_v[k_ // 4,
                                pl.ds((k_ % 4) * (CH // 2) + g * 16, 16)]
                    ev = vi << 16
                    od = vi & jnp.int32(-65536)
                    for lane in range(32):
                        src = ev if lane % 2 == 0 else od
                        w = lax.bitcast_convert_type(
                            src[lane // 2], jnp.float32)
                        sv = jnp.full((16,), w, jnp.float32)
                        e = g * 32 + lane
                        for j in range(F // 16):
                            sl = pl.ds(j * 16, 16)
                            gbuf[b, e, sl] = gbuf[b, e, sl] * sv
                    return cc

                lax.fori_loop(0, CH // 32, scale, 0)

                @pl.when(k_ + 2 < n_chunks)
                def _():
                    g_desc(k_ + 2, (b + 2) % NBUF).start()

                unpack_rows(k_, b)
                pltpu.sync_copy(gbuf.at[b], acc.at[rbuf.at[b]], add=True)
            return carry

        lax.fori_loop(0, n_chunks // NBUF, triple, 0)
        plsc.subcore_barrier()

        pltpu.sync_copy(acc.at[pl.ds(base, STRIPE)],
                        out_hbm.at[c, pl.ds(base, STRIPE)])

        @pl.when(s == 0)
        def _():
            pltpu.sync_copy(acc.at[pl.ds(NS * STRIPE, TAIL)],
                            out_hbm.at[c, pl.ds(NS * STRIPE, TAIL)])

    return k(support, cols3, rpair3, vals3)


def kernel(input, adj_indices, adj_values, weight, bias):
    support = _matmul(input, weight)

    rows = adj_indices[0].astype(jnp.int32)
    cols = adj_indices[1].astype(jnp.int32)
    vals = adj_values.astype(jnp.float32)

    n_edges = vals.shape[0]
    unit = 12 * CH
    per = -(-n_edges // (NW * unit)) * unit
    n_chunks = per // CH
    e_pad = per * NW
    pad = e_pad - n_edges
    rows = jnp.pad(rows, (0, pad))
    cols = jnp.pad(cols, (0, pad))
    vals = jnp.pad(vals, (0, pad))

    cols3 = cols.reshape(NW, n_chunks // 2, 2 * CH)
    r4 = rows.reshape(-1, 2, 2, 16)
    rpair = r4[:, :, 0, :] | (r4[:, :, 1, :] << 16)
    rpair3 = rpair.reshape(NW, n_chunks // 4, 2 * CH)
    vbits = jax.lax.bitcast_convert_type(
        vals.astype(jnp.bfloat16), jnp.uint16).astype(jnp.uint32)
    vpair = jax.lax.bitcast_convert_type(
        vbits[0::2] | (vbits[1::2] << 16), jnp.int32)
    vals3 = vpair.reshape(NW, n_chunks // 4, 2 * CH)

    parts = _sc_spmm(support, cols3, rpair3, vals3, n_chunks)
    return _combine(parts, bias.reshape(1, F))


# speedup vs baseline: 1.7012x; 1.7012x over previous
import functools

import jax
import jax.numpy as jnp
from jax import lax
from jax.experimental import pallas as pl
from jax.experimental.pallas import tpu as pltpu
from jax.experimental.pallas import tpu_sc as plsc

N_NODES = 10000
F = 128
CH = 64
NBUF = 3
NC = 2
NS = 16
NW = NC * NS
STRIPE = 624
TAIL = N_NODES - NS * STRIPE


def _mm_body(x_ref, w_ref, o_ref):
    o_ref[...] = jnp.dot(x_ref[...], w_ref[...],
                         preferred_element_type=jnp.float32)


def _matmul(x, w):
    m = x.shape[0]
    bm = 1000
    grid = (m // bm,)
    return pl.pallas_call(
        _mm_body,
        grid=grid,
        in_specs=[
            pl.BlockSpec((bm, F), lambda i: (i, 0)),
            pl.BlockSpec((F, F), lambda i: (0, 0)),
        ],
        out_specs=pl.BlockSpec((bm, F), lambda i: (i, 0)),
        out_shape=jax.ShapeDtypeStruct((m, F), jnp.float32),
    )(x, w)


def _combine_body(p_ref, b_ref, o_ref):
    o_ref[...] = p_ref[0] + p_ref[1] + b_ref[...]


def _combine(parts, bias2d):
    m = parts.shape[1]
    bm = 1000
    grid = (m // bm,)
    return pl.pallas_call(
        _combine_body,
        grid=grid,
        in_specs=[
            pl.BlockSpec((NC, bm, F), lambda i: (0, i, 0)),
            pl.BlockSpec((1, F), lambda i: (0, 0)),
        ],
        out_specs=pl.BlockSpec((bm, F), lambda i: (i, 0)),
        out_shape=jax.ShapeDtypeStruct((m, F), jnp.float32),
    )(parts, bias2d)


def _sc_spmm(support, combo3, n_chunks):
    mesh = plsc.VectorSubcoreMesh(core_axis_name="c", subcore_axis_name="s")

    @functools.partial(
        pl.kernel,
        mesh=mesh,
        out_type=jax.ShapeDtypeStruct((NC, N_NODES, F), jnp.float32),
        scratch_types=[
            pltpu.VMEM((n_chunks, 2 * CH), jnp.int32),
            pltpu.VMEM((NBUF, CH), jnp.int32),
            pltpu.VMEM((NBUF, CH, F), jnp.float32),
            pltpu.VMEM_SHARED((N_NODES, F), jnp.float32),
            pltpu.SemaphoreType.DMA,
            pltpu.SemaphoreType.DMA,
            pltpu.SemaphoreType.DMA,
        ],
    )
    def k(support_hbm, combo_hbm, out_hbm,
          combo_v, rbuf, gbuf, acc, sg0, sg1, sg2):
        c = lax.axis_index("c")
        s = lax.axis_index("s")
        wid = c * NS + s
        gsems = (sg0, sg1, sg2)

        pltpu.sync_copy(combo_hbm.at[wid], combo_v)

        zeros16 = jnp.zeros((16,), jnp.float32)

        def zrow(e, carry):
            for j in range(F // 16):
                gbuf[0, e, pl.ds(j * 16, 16)] = zeros16
            return carry

        lax.fori_loop(0, CH, zrow, 0)
        base = s * STRIPE
        for t in range(STRIPE // CH):
            pltpu.sync_copy(gbuf.at[0], acc.at[pl.ds(base + t * CH, CH)])
        rem = STRIPE % CH
        if rem:
            pltpu.sync_copy(gbuf.at[0, pl.ds(0, rem)],
                            acc.at[pl.ds(base + STRIPE - rem, rem)])

        @pl.when(s == 0)
        def _():
            pltpu.sync_copy(gbuf.at[0, pl.ds(0, TAIL)],
                            acc.at[pl.ds(NS * STRIPE, TAIL)])

        plsc.subcore_barrier()

        def g_desc(t, b):
            return pltpu.make_async_copy(
                support_hbm.at[combo_v.at[t, pl.ds(0, CH)]],
                gbuf.at[b], gsems[b])

        def unpack_rows(t, dst):
            for jg in range(2):
                p = combo_v[t, pl.ds(CH + jg * 16, 16)]
                rbuf[dst, pl.ds(jg * 32, 16)] = p & 0xFFFF
                rbuf[dst, pl.ds(jg * 32 + 16, 16)] = (
                    lax.shift_right_logical(p, 16))

        g_desc(0, 0).start()
        g_desc(1, 1).start()

        def triple(kk, carry):
            for b in range(NBUF):
                k_ = kk * NBUF + b
                g_desc(k_, b).wait()

                def scale(g, cc):
                    vi = combo_v[k_, pl.ds(CH + CH // 2 + g * 16, 16)]
                    ev = vi << 16
                    od = vi & jnp.int32(-65536)
                    for lane in range(32):
                        src = ev if lane % 2 == 0 else od
                        w = lax.bitcast_convert_type(
                            src[lane // 2], jnp.float32)
                        sv = jnp.full((16,), w, jnp.float32)
                        e = g * 32 + lane
                        for j in range(F // 16):
                            sl = pl.ds(j * 16, 16)
                            gbuf[b, e, sl] = gbuf[b, e, sl] * sv
                    return cc

                lax.fori_loop(0, CH // 32, scale, 0)

                @pl.when(k_ + 2 < n_chunks)
                def _():
                    g_desc(k_ + 2, (b + 2) % NBUF).start()

                unpack_rows(k_, b)
                pltpu.sync_copy(gbuf.at[b], acc.at[rbuf.at[b]], add=True)
            return carry

        lax.fori_loop(0, n_chunks // NBUF, triple, 0)
        plsc.subcore_barrier()

        pltpu.sync_copy(acc.at[pl.ds(base, STRIPE)],
                        out_hbm.at[c, pl.ds(base, STRIPE)])

        @pl.when(s == 0)
        def _():
            pltpu.sync_copy(acc.at[pl.ds(NS * STRIPE, TAIL)],
                            out_hbm.at[c, pl.ds(NS * STRIPE, TAIL)])

    return k(support, combo3)


def kernel(input, adj_indices, adj_values, weight, bias):
    support = _matmul(input, weight)

    rows = adj_indices[0].astype(jnp.int32)
    cols = adj_indices[1].astype(jnp.int32)
    vals = adj_values.astype(jnp.float32)

    n_edges = vals.shape[0]
    unit = NBUF * CH
    per = -(-n_edges // (NW * unit)) * unit
    n_chunks = per // CH
    e_pad = per * NW
    pad = e_pad - n_edges
    rows = jnp.pad(rows, (0, pad))
    cols = jnp.pad(cols, (0, pad))
    vals = jnp.pad(vals, (0, pad))

    colsc = cols.reshape(NW, n_chunks, CH)
    r4 = rows.reshape(NW, n_chunks, 2, 2, 16)
    rpair = r4[..., 0, :] | (r4[..., 1, :] << 16)
    rpair = rpair.reshape(NW, n_chunks, CH // 2)
    vbits = jax.lax.bitcast_convert_type(
        vals.astype(jnp.bfloat16), jnp.uint16).astype(jnp.uint32)
    vbits = vbits.reshape(NW, n_chunks, CH // 2, 2)
    vpair = jax.lax.bitcast_convert_type(
        vbits[..., 0] | (vbits[..., 1] << 16), jnp.int32)
    combo3 = jnp.concatenate([colsc, rpair, vpair], axis=-1)

    parts = _sc_spmm(support, combo3, n_chunks)
    return _combine(parts, bias.reshape(1, F))


# speedup vs baseline: 3.0872x; 1.8147x over previous
import functools

import jax
import jax.numpy as jnp
from jax import lax
from jax.experimental import pallas as pl
from jax.experimental.pallas import tpu as pltpu
from jax.experimental.pallas import tpu_sc as plsc

N_NODES = 10000
F = 128
CH = 128
NC = 2
NS = 16
NW = NC * NS
STRIPE = 624
TAIL = N_NODES - NS * STRIPE


def _mm_body(x_ref, w_ref, o_ref):
    o_ref[...] = jnp.dot(x_ref[...], w_ref[...],
                         preferred_element_type=jnp.float32)


def _matmul(x, w):
    m = x.shape[0]
    bm = 1000
    grid = (m // bm,)
    return pl.pallas_call(
        _mm_body,
        grid=grid,
        in_specs=[
            pl.BlockSpec((bm, F), lambda i: (i, 0)),
            pl.BlockSpec((F, F), lambda i: (0, 0)),
        ],
        out_specs=pl.BlockSpec((bm, F), lambda i: (i, 0)),
        out_shape=jax.ShapeDtypeStruct((m, F), jnp.float32),
    )(x, w)


def _combine_body(p_ref, b_ref, o_ref):
    o_ref[...] = p_ref[0] + p_ref[1] + b_ref[...]


def _combine(parts, bias2d):
    m = parts.shape[1]
    bm = 1000
    grid = (m // bm,)
    return pl.pallas_call(
        _combine_body,
        grid=grid,
        in_specs=[
            pl.BlockSpec((NC, bm, F), lambda i: (0, i, 0)),
            pl.BlockSpec((1, F), lambda i: (0, 0)),
        ],
        out_specs=pl.BlockSpec((bm, F), lambda i: (i, 0)),
        out_shape=jax.ShapeDtypeStruct((m, F), jnp.float32),
    )(parts, bias2d)


def _sc_spmm(support, rows3, cols3, vals3, n_chunks):
    mesh = plsc.VectorSubcoreMesh(core_axis_name="c", subcore_axis_name="s")

    @functools.partial(
        pl.kernel,
        mesh=mesh,
        out_type=jax.ShapeDtypeStruct((NC, N_NODES, F), jnp.float32),
        scratch_types=[
            pltpu.VMEM((n_chunks, CH), jnp.int32),
            pltpu.VMEM((n_chunks, CH), jnp.int32),
            pltpu.VMEM((n_chunks, CH), jnp.float32),
            pltpu.VMEM((CH, F), jnp.float32),
            pltpu.VMEM_SHARED((N_NODES, F), jnp.float32),
            pltpu.SemaphoreType.DMA,
        ],
    )
    def k(support_hbm, rows_hbm, cols_hbm, vals_hbm, out_hbm,
          cols_v, rows_v, vals_v, gbuf, acc, sem):
        c = lax.axis_index("c")
        s = lax.axis_index("s")
        wid = c * NS + s

        pltpu.sync_copy(rows_hbm.at[wid], rows_v)
        pltpu.sync_copy(cols_hbm.at[wid], cols_v)
        pltpu.sync_copy(vals_hbm.at[wid], vals_v)

        zeros16 = jnp.zeros((16,), jnp.float32)

        def zrow(e, carry):
            for j in range(F // 16):
                gbuf[e, pl.ds(j * 16, 16)] = zeros16
            return carry

        lax.fori_loop(0, CH, zrow, 0)
        base = s * STRIPE
        for t in range(STRIPE // CH):
            pltpu.sync_copy(gbuf, acc.at[pl.ds(base + t * CH, CH)])
        rem = STRIPE % CH
        if rem:
            pltpu.sync_copy(gbuf.at[pl.ds(0, rem)],
                            acc.at[pl.ds(base + STRIPE - rem, rem)])

        @pl.when(s == 0)
        def _():
            pltpu.sync_copy(gbuf.at[pl.ds(0, TAIL)],
                            acc.at[pl.ds(NS * STRIPE, TAIL)])

        plsc.subcore_barrier()

        def chunk(kk, carry):
            pltpu.async_copy(support_hbm.at[cols_v.at[kk]], gbuf, sem).wait()

            def scale(g, cc):
                vv = vals_v[kk, pl.ds(g * 16, 16)]
                for lane in range(16):
                    sv = jnp.full((16,), vv[lane], jnp.float32)
                    e = g * 16 + lane
                    for j in range(F // 16):
                        sl = pl.ds(j * 16, 16)
                        gbuf[e, sl] = gbuf[e, sl] * sv
                return cc

            lax.fori_loop(0, CH // 16, scale, 0)
            pltpu.sync_copy(gbuf, acc.at[rows_v.at[kk]], add=True)
            return carry

        lax.fori_loop(0, n_chunks, chunk, 0)
        plsc.subcore_barrier()

        pltpu.sync_copy(acc.at[pl.ds(base, STRIPE)],
                        out_hbm.at[c, pl.ds(base, STRIPE)])

        @pl.when(s == 0)
        def _():
            pltpu.sync_copy(acc.at[pl.ds(NS * STRIPE, TAIL)],
                            out_hbm.at[c, pl.ds(NS * STRIPE, TAIL)])

    return k(support, rows3, cols3, vals3)


def kernel(input, adj_indices, adj_values, weight, bias):
    support = _matmul(input, weight)

    rows = adj_indices[0].astype(jnp.int32)
    cols = adj_indices[1].astype(jnp.int32)
    vals = adj_values.astype(jnp.float32)

    n_edges = vals.shape[0]
    per = -(-n_edges // (NW * CH)) * CH
    n_chunks = per // CH
    e_pad = per * NW
    pad = e_pad - n_edges
    rows = jnp.pad(rows, (0, pad))
    cols = jnp.pad(cols, (0, pad))
    vals = jnp.pad(vals, (0, pad))
    rows3 = rows.reshape(NW, n_chunks, CH)
    cols3 = cols.reshape(NW, n_chunks, CH)
    vals3 = vals.reshape(NW, n_chunks, CH)

    parts = _sc_spmm(support, rows3, cols3, vals3, n_chunks)
    return _combine(parts, bias.reshape(1, F))
